# baseline (speedup 1.0000x reference)
import functools

import jax
import jax.numpy as jnp
from jax import lax
from jax.experimental import pallas as pl
from jax.experimental.pallas import tpu as pltpu
from jax.experimental.pallas import tpu_sc as plsc

N_NODES = 10000
N_EDGES = 320000
D = 128

NW = 32
NPB = 320
N_PAD = NW * NPB
TRASH = NPB
BLK = 128
FLUSH_AT = BLK - 16
EP = 371200
SCH = 2000

_mesh = plsc.VectorSubcoreMesh(core_axis_name="c", subcore_axis_name="s")
_sc_params = pltpu.CompilerParams(needs_layout_passes=False)


def _wid():
    return lax.axis_index("s") * 2 + lax.axis_index("c")


def _bin_offsets(nbuf, w):
    iota = lax.iota(jnp.int32, 16)
    v0 = plsc.load_gather(nbuf, [iota * 8])
    v1 = plsc.load_gather(nbuf, [iota * 8 + 128])
    c0 = plsc.cumsum(v0)
    c1 = plsc.cumsum(v1) + c0[15]
    my_nb = plsc.load_gather(nbuf, [jnp.full((16,), w * 8, jnp.int32)])[0]
    sel = jnp.where(jnp.full((16,), w, jnp.int32) < 16, c0, c1)
    lane = jnp.where(w < 16, w, w - 16)
    onehot = (iota == lane).astype(jnp.int32)
    my_incl = jnp.sum(sel * onehot)
    return my_incl - my_nb, my_nb


def _mm_pre_body(h_ref, w1_ref, b1_ref, a_ref, b_ref, *, input_relu):
    h = h_ref[...]
    if input_relu:
        h = jnp.maximum(h, 0.0)
    wa = w1_ref[0:D, :]
    wb = w1_ref[D : 2 * D, :]
    a_ref[...] = (
        jnp.dot(h, wa - wb, preferred_element_type=jnp.float32) + b1_ref[...]
    )
    b_ref[...] = jnp.dot(h, wb, preferred_element_type=jnp.float32)


def _mm_pre(h, w1, b1, input_relu):
    nb = 1280
    return pl.pallas_call(
        functools.partial(_mm_pre_body, input_relu=input_relu),
        grid=(N_PAD // nb,),
        in_specs=[
            pl.BlockSpec((nb, D), lambda i: (i, 0)),
            pl.BlockSpec((2 * D, D), lambda i: (0, 0)),
            pl.BlockSpec((1, D), lambda i: (0, 0)),
        ],
        out_specs=(
            pl.BlockSpec((nb, D), lambda i: (i, 0)),
            pl.BlockSpec((nb, D), lambda i: (i, 0)),
        ),
        out_shape=(
            jax.ShapeDtypeStruct((N_PAD, D), jnp.float32),
            jax.ShapeDtypeStruct((N_PAD, D), jnp.float32),
        ),
    )(h, w1, b1.reshape(1, D))


def _mm_edge_body(u_ref, w2_ref, b2_ref, m_ref):
    m_ref[...] = (
        jnp.dot(u_ref[...], w2_ref[...], preferred_element_type=jnp.float32)
        + b2_ref[...]
    )


def _mm_edge(u, w2, b2):
    eb = 2560
    return pl.pallas_call(
        _mm_edge_body,
        grid=(EP // eb,),
        in_specs=[
            pl.BlockSpec((eb, D), lambda i: (i, 0)),
            pl.BlockSpec((D, D), lambda i: (0, 0)),
            pl.BlockSpec((1, D), lambda i: (0, 0)),
        ],
        out_specs=pl.BlockSpec((eb, D), lambda i: (i, 0)),
        out_shape=jax.ShapeDtypeStruct((EP, D), jnp.float32),
    )(u, w2, b2.reshape(1, D))


def _mm_out_body(h_ref, wo_ref, bo_ref, o_ref):
    h = jnp.maximum(h_ref[...], 0.0)
    o_ref[...] = (
        jnp.dot(h, wo_ref[...], preferred_element_type=jnp.float32) + bo_ref[...]
    )


def _mm_out(h, wo, bo):
    return pl.pallas_call(
        _mm_out_body,
        out_shape=jax.ShapeDtypeStruct((h.shape[0], 1), jnp.float32),
    )(h, wo, bo.reshape(1, 1))


def _bin_body(dst_hbm, src_hbm, lists_hbm, lists2_hbm, nblk_hbm,
              dvec0, dvec1, svec0, svec1, ebuf, sbuf, nbv,
              sem_d0, sem_d1, sem_s0, sem_s1):
    w = _wid()
    lo = w * NPB
    hi = lo + NPB
    iota = lax.iota(jnp.int32, 16)
    trash = jnp.full((16,), TRASH, jnp.int32)
    dv = (dvec0, dvec1)
    sv = (svec0, svec1)
    sems = ((sem_d0, sem_s0), (sem_d1, sem_s1))

    for k in range(BLK // 16):
        ebuf[pl.ds(k * 16, 16)] = trash
        sbuf[pl.ds(k * 16, 16)] = jnp.zeros((16,), jnp.int32)

    def issue(ci, p):
        pltpu.async_copy(dst_hbm.at[pl.ds(ci * SCH, SCH)], dv[p], sems[p][0])
        pltpu.async_copy(src_hbm.at[pl.ds(ci * SCH, SCH)], sv[p], sems[p][1])

    def wait(p):
        pltpu.make_async_copy(dst_hbm.at[pl.ds(0, SCH)], dv[p], sems[p][0]).wait()
        pltpu.make_async_copy(src_hbm.at[pl.ds(0, SCH)], sv[p], sems[p][1]).wait()

    def flush(off_nb):
        off, nb = off_nb
        pltpu.sync_copy(ebuf, lists_hbm.at[w, pl.ds(nb * BLK, BLK)])
        pltpu.sync_copy(sbuf, lists2_hbm.at[w, pl.ds(nb * BLK, BLK)])
        return (jnp.int32(0), nb + 1)

    def do_chunk(ci, p, carry):
        wait(p)

        def vec(i, carry):
            off, nb = carry
            v = dv[p][pl.ds(i * 16, 16)]
            s = sv[p][pl.ds(i * 16, 16)]
            m = (v >= lo) & (v < hi)
            eids = ci * SCH + i * 16 + iota
            packed = jnp.where(m, eids * 512 + (v - lo), trash)
            key = jnp.where(m, jnp.int32(0), jnp.int32(1))
            _, spacked = plsc.sort_key_val(key, packed)
            _, ssrc = plsc.sort_key_val(key, jnp.where(m, s, 0))
            ebuf[pl.ds(off, 16)] = spacked
            sbuf[pl.ds(off, 16)] = ssrc
            off = off + jnp.sum(m.astype(jnp.int32))
            return lax.cond(off >= FLUSH_AT, flush, lambda c: c, (off, nb))

        return lax.fori_loop(0, SCH // 16, vec, carry)

    n_chunks = N_EDGES // SCH
    issue(0, 0)

    def super_chunk(s, carry):
        ci = s * 2

        @pl.when(ci + 1 < n_chunks)
        def _():
            issue(ci + 1, 1)

        carry0 = do_chunk(ci, 0, carry)

        @pl.when(ci + 2 < n_chunks)
        def _():
            issue(ci + 2, 0)

        return do_chunk(ci + 1, 1, carry0)

    carry = lax.fori_loop(0, n_chunks // 2, super_chunk,
                          (jnp.int32(0), jnp.int32(0)))
    _, nb = flush(carry)
    nbv[...] = jnp.full((16,), nb, jnp.int32)
    pltpu.sync_copy(nbv.at[pl.ds(0, 8)], nblk_hbm.at[pl.ds(w * 8, 8)])


def _bin_edges(dst, src):
    f = pl.kernel(
        _bin_body,
        out_type=(
            jax.ShapeDtypeStruct((NW, N_EDGES), jnp.int32),
            jax.ShapeDtypeStruct((NW, N_EDGES), jnp.int32),
            jax.ShapeDtypeStruct((NW * 8,), jnp.int32),
        ),
        mesh=_mesh,
        compiler_params=_sc_params,
        scratch_types=[
            pltpu.VMEM((SCH,), jnp.int32),
            pltpu.VMEM((SCH,), jnp.int32),
            pltpu.VMEM((SCH,), jnp.int32),
            pltpu.VMEM((SCH,), jnp.int32),
            pltpu.VMEM((BLK,), jnp.int32),
            pltpu.VMEM((BLK,), jnp.int32),
            pltpu.VMEM((16,), jnp.int32),
            pltpu.SemaphoreType.DMA,
            pltpu.SemaphoreType.DMA,
            pltpu.SemaphoreType.DMA,
            pltpu.SemaphoreType.DMA,
        ],
    )
    return f(dst, src)


def _uperm_body(a_hbm, b_hbm, lists_hbm, lists2_hbm, nblk_hbm, u_hbm,
                awin, pbuf0, pbuf1, s2buf0, s2buf1, s2buf2, s2buf3,
                dl0, dl1, brows0, brows1, nbuf,
                sem_l0, sem_l1, sem_s0, sem_s1, sem_s2, sem_s3,
                sem_b0, sem_b1, sem_w0, sem_w1):
    w = _wid()
    lo = w * NPB
    pltpu.sync_copy(a_hbm.at[pl.ds(lo, NPB)], awin.at[pl.ds(0, NPB)])
    pltpu.sync_copy(nblk_hbm, nbuf)
    boff, nb = _bin_offsets(nbuf, w)
    pb = (pbuf0, pbuf1)
    s2 = (s2buf0, s2buf1, s2buf2, s2buf3)
    dl = (dl0, dl1)
    br = (brows0, brows1)
    sl = (sem_l0, sem_l1)
    ss = (sem_s0, sem_s1, sem_s2, sem_s3)
    sb = (sem_b0, sem_b1)
    sw = (sem_w0, sem_w1)

    def issue_list(bi, p, q):
        pltpu.async_copy(lists_hbm.at[w, pl.ds(bi * BLK, BLK)], pb[p], sl[p])
        pltpu.async_copy(lists2_hbm.at[w, pl.ds(bi * BLK, BLK)], s2[q], ss[q])

    def start_gather(bi, p, q):
        pltpu.make_async_copy(lists_hbm.at[w, pl.ds(0, BLK)], pb[p], sl[p]).wait()
        pltpu.make_async_copy(lists_hbm.at[w, pl.ds(0, BLK)], s2[q], ss[q]).wait()
        for k in range(BLK // 16):
            v = pb[p][pl.ds(k * 16, 16)]
            dl[p][pl.ds(k * 16, 16)] = jnp.bitwise_and(v, 511)
        for j in range(4):
            pltpu.async_copy(b_hbm.at[s2[q].at[pl.ds(j * 32, 32)]],
                             br[p].at[pl.ds(j * 32, 32)], sb[p])

    def finish_block(bi, p, q):
        for j in range(4):
            pltpu.make_async_copy(b_hbm.at[s2[q].at[pl.ds(j * 32, 32)]],
                                  br[p].at[pl.ds(j * 32, 32)], sb[p]).wait()

        def row(r, c2):
            d = dl[p][pl.ds(r, 16)][0]
            for ch in range(D // 16):
                va = awin[d, pl.ds(ch * 16, 16)]
                vb = br[p][r, pl.ds(ch * 16, 16)]
                br[p][r, pl.ds(ch * 16, 16)] = jnp.maximum(va + vb, 0.0)
            return c2

        lax.fori_loop(0, BLK, row, 0)
        pltpu.async_copy(br[p], u_hbm.at[pl.ds((boff + bi) * BLK, BLK)], sw[p])

    def wait_write(p):
        pltpu.make_async_copy(br[p], u_hbm.at[pl.ds(0, BLK)], sw[p]).wait()

    @pl.when(nb > 0)
    def _():
        issue_list(0, 0, 0)

    def step(i, p, q):
        @pl.when(i + 1 < nb)
        def _():
            issue_list(i + 1, 1 - p, (q + 1) % 4)

        @pl.when(i < nb)
        def _():
            @pl.when(i >= 2)
            def _():
                wait_write(p)

            start_gather(i, p, q)

        @pl.when((i >= 1) & (i <= nb))
        def _():
            finish_block(i - 1, 1 - p, (q + 3) % 4)

    def super_step(s, c):
        for k in range(4):
            step(s * 4 + k, k % 2, k)
        return c

    lax.fori_loop(0, (nb + 4) // 4, super_step, 0)
    @pl.when(nb >= 2)
    def _():
        wait_write(0)
        wait_write(1)

    @pl.when(nb == 1)
    def _():
        wait_write(0)


def _uperm(a, b, lists, lists2, nblk):
    f = pl.kernel(
        _uperm_body,
        out_type=jax.ShapeDtypeStruct((EP, D), jnp.float32),
        mesh=_mesh,
        compiler_params=_sc_params,
        scratch_types=(
            [pltpu.VMEM((NPB + 1, D), jnp.float32)]
            + [pltpu.VMEM((BLK,), jnp.int32)] * 6
            + [pltpu.VMEM((BLK + 16,), jnp.int32)] * 2
            + [pltpu.VMEM((BLK, D), jnp.float32)] * 2
            + [pltpu.VMEM((NW * 8,), jnp.int32)]
            + [pltpu.SemaphoreType.DMA] * 10
        ),
    )
    return f(a, b, lists, lists2, nblk)


def _scatter_body(m_hbm, lists_hbm, nblk_hbm, agg_hbm,
                  acc, mrows0, mrows1, dbuf0, dbuf1, pbuf0, pbuf1, nbuf,
                  sem_l0, sem_l1, sem_m0, sem_m1):
    w = _wid()
    neg_inf = jnp.full((16,), -jnp.inf, jnp.float32)
    mr = (mrows0, mrows1)
    db = (dbuf0, dbuf1)
    pb = (pbuf0, pbuf1)
    sl = (sem_l0, sem_l1)
    sm = (sem_m0, sem_m1)

    def init(r, c):
        for ch in range(D // 16):
            acc[r, pl.ds(ch * 16, 16)] = neg_inf
        return c

    lax.fori_loop(0, NPB + 1, init, 0)

    pltpu.sync_copy(nblk_hbm, nbuf)
    boff, nb = _bin_offsets(nbuf, w)

    def issue(bi, p):
        pltpu.async_copy(lists_hbm.at[w, pl.ds(bi * BLK, BLK)], pb[p], sl[p])
        pltpu.async_copy(m_hbm.at[pl.ds((boff + bi) * BLK, BLK)], mr[p], sm[p])

    def process(bi, p):
        pltpu.make_async_copy(lists_hbm.at[w, pl.ds(0, BLK)], pb[p], sl[p]).wait()
        pltpu.make_async_copy(m_hbm.at[pl.ds(0, BLK)], mr[p], sm[p]).wait()
        for k in range(BLK // 16):
            v = pb[p][pl.ds(k * 16, 16)]
            db[p][pl.ds(k * 16, 16)] = jnp.bitwise_and(v, 511)

        def row(r, c2):
            d = db[p][pl.ds(r, 16)][0]
            for ch in range(D // 16):
                cur = acc[d, pl.ds(ch * 16, 16)]
                mv = mr[p][r, pl.ds(ch * 16, 16)]
                acc[d, pl.ds(ch * 16, 16)] = jnp.maximum(cur, mv)
            return c2

        lax.fori_loop(0, BLK, row, 0)

    @pl.when(nb > 0)
    def _():
        issue(0, 0)

    def step(i, p):
        @pl.when(i + 1 < nb)
        def _():
            issue(i + 1, 1 - p)

        @pl.when(i < nb)
        def _():
            process(i, p)

    def super_step(s, c):
        step(s * 2, 0)
        step(s * 2 + 1, 1)
        return c

    lax.fori_loop(0, (nb + 1) // 2, super_step, 0)

    def fin(r, c):
        for ch in range(D // 16):
            v = acc[r, pl.ds(ch * 16, 16)]
            acc[r, pl.ds(ch * 16, 16)] = jnp.where(v == -jnp.inf, 0.0, v)
        return c

    lax.fori_loop(0, NPB, fin, 0)
    pltpu.sync_copy(acc.at[pl.ds(0, NPB)],
                    agg_hbm.at[pl.ds(w * NPB, NPB)])


def _scatter_max(m, lists, nblk):
    f = pl.kernel(
        _scatter_body,
        out_type=jax.ShapeDtypeStruct((N_PAD, D), jnp.float32),
        mesh=_mesh,
        compiler_params=_sc_params,
        scratch_types=(
            [pltpu.VMEM((NPB + 1, D), jnp.float32)]
            + [pltpu.VMEM((BLK, D), jnp.float32)] * 2
            + [pltpu.VMEM((BLK + 16,), jnp.int32)] * 2
            + [pltpu.VMEM((BLK,), jnp.int32)] * 2
            + [pltpu.VMEM((NW * 8,), jnp.int32)]
            + [pltpu.SemaphoreType.DMA] * 4
        ),
    )
    return f(m, lists, nblk)


def kernel(x, edge_index, W1_0, b1_0, W2_0, b2_0, W1_1, b1_1, W2_1, b2_1, Wo, bo):
    src = edge_index[0].astype(jnp.int32)
    dst = edge_index[1].astype(jnp.int32)

    lists, lists2, nblk = _bin_edges(dst, src)

    a0, b0 = _mm_pre(x, W1_0, b1_0, input_relu=False)
    u0 = _uperm(a0, b0, lists, lists2, nblk)
    m0 = _mm_edge(u0, W2_0, b2_0)
    agg0 = _scatter_max(m0, lists, nblk)

    a1, b1 = _mm_pre(agg0[:N_NODES], W1_1, b1_1, input_relu=True)
    u1 = _uperm(a1, b1, lists, lists2, nblk)
    m1 = _mm_edge(u1, W2_1, b2_1)
    agg1 = _scatter_max(m1, lists, nblk)[:N_NODES]

    out = _mm_out(agg1, Wo, bo)
    return out.squeeze(-1)


# speedup vs baseline: 1.2007x; 1.2007x over previous
import functools

import jax
import jax.numpy as jnp
from jax import lax
from jax.experimental import pallas as pl
from jax.experimental.pallas import tpu as pltpu
from jax.experimental.pallas import tpu_sc as plsc

N_NODES = 10000
N_EDGES = 320000
D = 128

NW = 32
NPB = 320
N_PAD = NW * NPB
TRASH = NPB
BLK = 128
FLUSH_AT = BLK - 16
GCH = 80
SCH = 2000

_mesh = plsc.VectorSubcoreMesh(core_axis_name="c", subcore_axis_name="s")
_sc_params = pltpu.CompilerParams(needs_layout_passes=False)


def _wid():
    return lax.axis_index("s") * 2 + lax.axis_index("c")


def _mm_pre_body(h_ref, w1_ref, b1_ref, a_ref, b_ref, *, input_relu):
    h = h_ref[...]
    if input_relu:
        h = jnp.maximum(h, 0.0)
    wa = w1_ref[0:D, :]
    wb = w1_ref[D : 2 * D, :]
    a_ref[...] = (
        jnp.dot(h, wa - wb, preferred_element_type=jnp.float32) + b1_ref[...]
    )
    b_ref[...] = jnp.dot(h, wb, preferred_element_type=jnp.float32)


def _mm_pre(h, w1, b1, input_relu):
    n = h.shape[0]
    return pl.pallas_call(
        functools.partial(_mm_pre_body, input_relu=input_relu),
        out_shape=(
            jax.ShapeDtypeStruct((n, D), jnp.float32),
            jax.ShapeDtypeStruct((n, D), jnp.float32),
        ),
    )(h, w1, b1.reshape(1, D))


def _mm_edge_body(u_ref, w2_ref, b2_ref, m_ref):
    m_ref[...] = (
        jnp.dot(u_ref[...], w2_ref[...], preferred_element_type=jnp.float32)
        + b2_ref[...]
    )


def _mm_edge(u, w2, b2):
    eb = 2560
    grid = N_EDGES // eb
    return pl.pallas_call(
        _mm_edge_body,
        grid=(grid,),
        in_specs=[
            pl.BlockSpec((eb, D), lambda i: (i, 0)),
            pl.BlockSpec((D, D), lambda i: (0, 0)),
            pl.BlockSpec((1, D), lambda i: (0, 0)),
        ],
        out_specs=pl.BlockSpec((eb, D), lambda i: (i, 0)),
        out_shape=jax.ShapeDtypeStruct((N_EDGES, D), jnp.float32),
    )(u, w2, b2.reshape(1, D))


def _mm_out_body(h_ref, wo_ref, bo_ref, o_ref):
    h = jnp.maximum(h_ref[...], 0.0)
    o_ref[...] = (
        jnp.dot(h, wo_ref[...], preferred_element_type=jnp.float32) + bo_ref[...]
    )


def _mm_out(h, wo, bo):
    return pl.pallas_call(
        _mm_out_body,
        out_shape=jax.ShapeDtypeStruct((h.shape[0], 1), jnp.float32),
    )(h, wo, bo.reshape(1, 1))


def _gather_body(a_hbm, b_hbm, src_hbm, dst_hbm, u_hbm,
                 didx, sidx, arows, brows, sem_a, sem_b):
    w = _wid()
    base = w * (N_EDGES // NW)
    n_chunks = (N_EDGES // NW) // GCH

    def chunk(ci, carry):
        off = base + ci * GCH
        pltpu.sync_copy(dst_hbm.at[pl.ds(off, GCH)], didx)
        pltpu.sync_copy(src_hbm.at[pl.ds(off, GCH)], sidx)
        ca = pltpu.async_copy(a_hbm.at[didx], arows, sem_a)
        cb = pltpu.async_copy(b_hbm.at[sidx], brows, sem_b)
        ca.wait()
        cb.wait()

        def row(r, c2):
            for c in range(D // 16):
                va = arows[r, pl.ds(c * 16, 16)]
                vb = brows[r, pl.ds(c * 16, 16)]
                arows[r, pl.ds(c * 16, 16)] = jnp.maximum(va + vb, 0.0)
            return c2

        lax.fori_loop(0, GCH, row, 0)
        pltpu.sync_copy(arows, u_hbm.at[pl.ds(off, GCH)])
        return carry

    lax.fori_loop(0, n_chunks, chunk, 0)


def _gather_add_relu(a, b, src, dst):
    f = pl.kernel(
        _gather_body,
        out_type=jax.ShapeDtypeStruct((N_EDGES, D), jnp.float32),
        mesh=_mesh,
        compiler_params=_sc_params,
        scratch_types=[
            pltpu.VMEM((GCH,), jnp.int32),
            pltpu.VMEM((GCH,), jnp.int32),
            pltpu.VMEM((GCH, D), jnp.float32),
            pltpu.VMEM((GCH, D), jnp.float32),
            pltpu.SemaphoreType.DMA,
            pltpu.SemaphoreType.DMA,
        ],
    )
    return f(a, b, src, dst)


def _bin_body(dst_hbm, lists_hbm, nblk_hbm, dvec, ebuf, nbv):
    w = _wid()
    lo = w * NPB
    hi = lo + NPB
    iota = lax.iota(jnp.int32, 16)
    trash = jnp.full((16,), TRASH, jnp.int32)

    for k in range(BLK // 16):
        ebuf[pl.ds(k * 16, 16)] = trash

    def flush(off_nb):
        off, nb = off_nb
        pltpu.sync_copy(ebuf, lists_hbm.at[w, pl.ds(nb * BLK, BLK)])
        return (jnp.int32(0), nb + 1)

    def chunk(ci, carry):
        pltpu.sync_copy(dst_hbm.at[pl.ds(ci * SCH, SCH)], dvec)

        def vec(i, carry):
            off, nb = carry
            v = dvec[pl.ds(i * 16, 16)]
            m = (v >= lo) & (v < hi)
            eids = ci * SCH + i * 16 + iota
            packed = jnp.where(m, eids * 512 + (v - lo), trash)
            key = jnp.where(m, jnp.int32(0), jnp.int32(1))
            _, spacked = plsc.sort_key_val(key, packed)
            ebuf[pl.ds(off, 16)] = spacked
            off = off + jnp.sum(m.astype(jnp.int32))
            return lax.cond(off >= FLUSH_AT, flush, lambda c: c, (off, nb))

        return lax.fori_loop(0, SCH // 16, vec, carry)

    carry = lax.fori_loop(0, N_EDGES // SCH, chunk, (jnp.int32(0), jnp.int32(0)))
    _, nb = flush(carry)
    nbv[...] = jnp.full((16,), nb, jnp.int32)
    pltpu.sync_copy(nbv.at[pl.ds(0, 8)], nblk_hbm.at[pl.ds(w * 8, 8)])


def _bin_edges(dst):
    f = pl.kernel(
        _bin_body,
        out_type=(
            jax.ShapeDtypeStruct((NW, N_EDGES), jnp.int32),
            jax.ShapeDtypeStruct((NW * 8,), jnp.int32),
        ),
        mesh=_mesh,
        compiler_params=_sc_params,
        scratch_types=[
            pltpu.VMEM((SCH,), jnp.int32),
            pltpu.VMEM((BLK,), jnp.int32),
            pltpu.VMEM((16,), jnp.int32),
        ],
    )
    return f(dst)


def _scatter_body(m_hbm, lists_hbm, nblk_hbm, agg_hbm,
                  acc, mrows, gidx, dbuf, pbuf, nbv, sem):
    w = _wid()
    neg_inf = jnp.full((16,), -jnp.inf, jnp.float32)

    def init(r, c):
        for ch in range(D // 16):
            acc[r, pl.ds(ch * 16, 16)] = neg_inf
        return c

    lax.fori_loop(0, NPB + 1, init, 0)

    pltpu.sync_copy(nblk_hbm.at[pl.ds(w * 8, 8)], nbv.at[pl.ds(0, 8)])
    nb = nbv[...][0]

    def blk(bi, c):
        pltpu.sync_copy(lists_hbm.at[w, pl.ds(bi * BLK, BLK)], pbuf)
        for k in range(BLK // 16):
            v = pbuf[pl.ds(k * 16, 16)]
            gidx[pl.ds(k * 16, 16)] = jnp.right_shift(v, 9)
            dbuf[pl.ds(k * 16, 16)] = jnp.bitwise_and(v, 511)
        pltpu.async_copy(m_hbm.at[gidx], mrows, sem).wait()

        def row(r, c2):
            d = dbuf[pl.ds(r, 16)][0]
            for ch in range(D // 16):
                cur = acc[d, pl.ds(ch * 16, 16)]
                mv = mrows[r, pl.ds(ch * 16, 16)]
                acc[d, pl.ds(ch * 16, 16)] = jnp.maximum(cur, mv)
            return c2

        lax.fori_loop(0, BLK, row, 0)
        return c

    lax.fori_loop(0, nb, blk, 0)

    def fin(r, c):
        for ch in range(D // 16):
            v = acc[r, pl.ds(ch * 16, 16)]
            acc[r, pl.ds(ch * 16, 16)] = jnp.where(v == -jnp.inf, 0.0, v)
        return c

    lax.fori_loop(0, NPB, fin, 0)
    pltpu.sync_copy(acc.at[pl.ds(0, NPB)],
                    agg_hbm.at[pl.ds(w * NPB, NPB)])


def _scatter_max(m, lists, nblk):
    f = pl.kernel(
        _scatter_body,
        out_type=jax.ShapeDtypeStruct((N_PAD, D), jnp.float32),
        mesh=_mesh,
        compiler_params=_sc_params,
        scratch_types=[
            pltpu.VMEM((NPB + 1, D), jnp.float32),
            pltpu.VMEM((BLK, D), jnp.float32),
            pltpu.VMEM((BLK,), jnp.int32),
            pltpu.VMEM((BLK + 16,), jnp.int32),
            pltpu.VMEM((BLK,), jnp.int32),
            pltpu.VMEM((16,), jnp.int32),
            pltpu.SemaphoreType.DMA,
        ],
    )
    return f(m, lists, nblk)


def kernel(x, edge_index, W1_0, b1_0, W2_0, b2_0, W1_1, b1_1, W2_1, b2_1, Wo, bo):
    src = edge_index[0].astype(jnp.int32)
    dst = edge_index[1].astype(jnp.int32)

    lists, nblk = _bin_edges(dst)

    a0, b0 = _mm_pre(x, W1_0, b1_0, input_relu=False)
    u0 = _gather_add_relu(a0, b0, src, dst)
    m0 = _mm_edge(u0, W2_0, b2_0)
    agg0 = _scatter_max(m0, lists, nblk)[:N_NODES]

    a1, b1 = _mm_pre(agg0, W1_1, b1_1, input_relu=True)
    u1 = _gather_add_relu(a1, b1, src, dst)
    m1 = _mm_edge(u1, W2_1, b2_1)
    agg1 = _scatter_max(m1, lists, nblk)[:N_NODES]

    out = _mm_out(agg1, Wo, bo)
    return out.squeeze(-1)
